# baseline (speedup 1.0000x reference)
import functools

import jax
import jax.numpy as jnp
from jax import lax
from jax.experimental import pallas as pl
from jax.experimental.pallas import tpu as pltpu
from jax.experimental.pallas import tpu_sc as plsc

RQ = 5
RQS = 2


def _gather_kernel(E, D, NW, CH, C):
  mesh = plsc.VectorSubcoreMesh(core_axis_name="c", subcore_axis_name="s")
  EW = CH * C

  @functools.partial(
      pl.kernel,
      out_type=[
          jax.ShapeDtypeStruct((E, D), jnp.float32),
          jax.ShapeDtypeStruct((E, D), jnp.float32),
      ],
      mesh=mesh,
      scratch_types=[
          pltpu.VMEM((CH, C), jnp.int32),
          pltpu.VMEM((CH, C), jnp.int32),
          pltpu.VMEM((RQ, C, D), jnp.float32),
          pltpu.VMEM((RQ, C, D), jnp.float32),
      ] + [pltpu.SemaphoreType.DMA] * (2 * RQ),
  )
  def k(x_hbm, col_hbm, row_hbm, gc_hbm, gr_hbm, idxc, idxr, bufc, bufr,
        *sems):
    sem_g = sems[:RQ]
    sem_w = sems[RQ:]
    cid = lax.axis_index("c")
    sid = lax.axis_index("s")
    wid = sid * 2 + cid
    base = wid * EW
    pltpu.sync_copy(col_hbm.at[wid], idxc)
    pltpu.sync_copy(row_hbm.at[wid], idxr)

    def fire(chunk, b):
      pltpu.async_copy(x_hbm.at[idxc.at[chunk]], bufc.at[b], sem_g[b])
      pltpu.async_copy(x_hbm.at[idxr.at[chunk]], bufr.at[b], sem_g[b])

    def gc_dst(chunk):
      return gc_hbm.at[pl.ds(base + chunk * C, C)]

    def gr_dst(chunk):
      return gr_hbm.at[pl.ds(base + chunk * C, C)]

    def drain(chunk, b):
      pltpu.make_async_copy(x_hbm.at[idxc.at[chunk]], bufc.at[b],
                            sem_g[b]).wait()
      pltpu.make_async_copy(x_hbm.at[idxr.at[chunk]], bufr.at[b],
                            sem_g[b]).wait()
      pltpu.async_copy(bufc.at[b], gc_dst(chunk), sem_w[b])
      pltpu.async_copy(bufr.at[b], gr_dst(chunk), sem_w[b])

    def wait_writes(chunk, b):
      pltpu.make_async_copy(bufc.at[b], gc_dst(chunk), sem_w[b]).wait()
      pltpu.make_async_copy(bufr.at[b], gr_dst(chunk), sem_w[b]).wait()

    for b in range(RQ):
      fire(b, b)

    @pl.loop(0, CH - RQ, step=RQ)
    def _(i):
      for b in range(RQ):
        drain(i + b, b)
      for b in range(RQ):
        wait_writes(i + b, b)
        fire(i + b + RQ, b)

    for b in range(RQ):
      drain(CH - RQ + b, b)
    for b in range(RQ):
      wait_writes(CH - RQ + b, b)

  return k


def _scatter_kernel(E, D, NP, NW, CH, C):
  mesh = plsc.VectorSubcoreMesh(core_axis_name="c", subcore_axis_name="s")
  EW = CH * C
  NS = 16
  rows_per_sub = NP // NS

  @functools.partial(
      pl.kernel,
      out_type=jax.ShapeDtypeStruct((2, NP, D), jnp.float32),
      mesh=mesh,
      scratch_types=[
          pltpu.VMEM((CH, C), jnp.int32),
          pltpu.VMEM((RQS, C, D), jnp.float32),
          pltpu.VMEM_SHARED((NP, D), jnp.float32),
      ] + [pltpu.SemaphoreType.DMA] * (2 * RQS),
  )
  def k(msg_hbm, row_hbm, init_hbm, part_hbm, idx, buf, acc, *sems):
    sem_l = sems[:RQS]
    sem_s = sems[RQS:]
    cid = lax.axis_index("c")
    sid = lax.axis_index("s")
    wid = sid * 2 + cid
    base = wid * EW
    pltpu.sync_copy(init_hbm.at[cid, pl.ds(sid * rows_per_sub, rows_per_sub)],
                    acc.at[pl.ds(sid * rows_per_sub, rows_per_sub)])
    pltpu.sync_copy(row_hbm.at[wid], idx)
    plsc.subcore_barrier()

    def fire(chunk, b):
      pltpu.async_copy(msg_hbm.at[pl.ds(base + chunk * C, C)], buf.at[b],
                       sem_l[b])

    def drain(chunk, b):
      pltpu.make_async_copy(msg_hbm.at[pl.ds(base + chunk * C, C)], buf.at[b],
                            sem_l[b]).wait()
      pltpu.async_copy(buf.at[b], acc.at[idx.at[chunk]], sem_s[b], add=True)

    def wait_add(chunk, b):
      pltpu.make_async_copy(buf.at[b], acc.at[idx.at[chunk]], sem_s[b]).wait()

    for b in range(RQS):
      fire(b, b)

    @pl.loop(0, CH - RQS, step=RQS)
    def _(i):
      for b in range(RQS):
        drain(i + b, b)
      for b in range(RQS):
        wait_add(i + b, b)
        fire(i + b + RQS, b)

    for b in range(RQS):
      drain(CH - RQS + b, b)
    for b in range(RQS):
      wait_add(CH - RQS + b, b)

    plsc.subcore_barrier()
    pltpu.sync_copy(acc.at[pl.ds(sid * rows_per_sub, rows_per_sub)],
                    part_hbm.at[cid, pl.ds(sid * rows_per_sub, rows_per_sub)])

  return k


def _tc_edge_body(gc_ref, gr_ref, offt2_ref, wb_ref, out_ref):
  xc = gc_ref[...]
  diff = xc - gr_ref[...]
  offt2 = offt2_ref[...]
  on2 = 0.25 * jnp.sum(offt2 * offt2, axis=0, keepdims=True)
  ones_m = jnp.ones((diff.shape[1], offt2.shape[1]), jnp.float32)
  t1 = lax.dot_general(diff * diff, ones_m, (((1,), (0,)), ((), ())),
                       preferred_element_type=jnp.float32)
  t2 = lax.dot_general(diff, offt2, (((1,), (0,)), ((), ())),
                       preferred_element_type=jnp.float32)
  d2 = jnp.maximum(t1 + t2 + on2, 0.0)
  sim = -jnp.sqrt(d2)
  sim = sim - jnp.max(sim, axis=1, keepdims=True)
  e = jnp.exp(sim)
  alpha = e / jnp.sum(e, axis=1, keepdims=True)
  mix = lax.dot_general(alpha, wb_ref[...], (((1,), (0,)), ((), ())),
                        preferred_element_type=jnp.float32)
  Dm = xc.shape[1]
  out_ref[...] = mix[:, :Dm] * xc + mix[:, Dm:]


def _combine_body(p_ref, o_ref):
  o_ref[...] = p_ref[0] + p_ref[1]


@jax.jit
def kernel(x, edge_index, offset, weight, bias):
  N, D = x.shape
  K = offset.shape[0]
  E = edge_index.shape[1]
  NW = 32
  CG = 80
  CS = 40
  G = 5
  assert E % (G * NW * CG) == 0 and E % (G * NW * CS) == 0
  ES = E // G
  CHG = ES // (NW * CG)
  CHS = ES // (NW * CS)
  assert CHG % RQ == 0 and CHG >= 2 * RQ
  assert CHS % RQS == 0 and CHS >= 2 * RQS

  row = edge_index[0]
  col = edge_index[1]
  row_g = row.reshape(G, NW, CHG, CG)
  col_g = col.reshape(G, NW, CHG, CG)
  row_s = row.reshape(G, NW, CHS, CS)

  offt2 = -2.0 * offset.T
  wb = jnp.concatenate([weight, bias], axis=1)
  BE = 1000
  assert ES % BE == 0
  NP = ((N + 1279) // 1280) * 1280

  gather_fn = _gather_kernel(ES, D, NW, CHG, CG)
  scatter_fn = _scatter_kernel(ES, D, NP, NW, CHS, CS)
  tc_fn = pl.pallas_call(
      _tc_edge_body,
      grid=(ES // BE,),
      in_specs=[
          pl.BlockSpec((BE, D), lambda i: (i, 0)),
          pl.BlockSpec((BE, D), lambda i: (i, 0)),
          pl.BlockSpec((D, K), lambda i: (0, 0)),
          pl.BlockSpec((K, 2 * D), lambda i: (0, 0)),
      ],
      out_specs=pl.BlockSpec((BE, D), lambda i: (i, 0)),
      out_shape=jax.ShapeDtypeStruct((ES, D), jnp.float32),
  )

  parts = jnp.zeros((2, NP, D), jnp.float32)
  for s in range(G):
    gc, gr = gather_fn(x, col_g[s], row_g[s])
    msg = tc_fn(gc, gr, offt2, wb)
    parts = scatter_fn(msg, row_s[s], parts)

  BN = 1280
  assert NP % BN == 0
  out = pl.pallas_call(
      _combine_body,
      grid=(NP // BN,),
      in_specs=[pl.BlockSpec((2, BN, D), lambda i: (0, i, 0))],
      out_specs=pl.BlockSpec((BN, D), lambda i: (i, 0)),
      out_shape=jax.ShapeDtypeStruct((NP, D), jnp.float32),
  )(parts)
  return out[:N]


# speedup vs baseline: 1.0397x; 1.0397x over previous
import functools

import jax
import jax.numpy as jnp
from jax import lax
from jax.experimental import pallas as pl
from jax.experimental.pallas import tpu as pltpu
from jax.experimental.pallas import tpu_sc as plsc

RQ = 5
RQS = 2


def _gather_kernel(E, D, NW, CH, C):
  mesh = plsc.VectorSubcoreMesh(core_axis_name="c", subcore_axis_name="s")
  EW = CH * C

  @functools.partial(
      pl.kernel,
      out_type=[
          jax.ShapeDtypeStruct((E, D), jnp.float32),
          jax.ShapeDtypeStruct((E, D), jnp.float32),
      ],
      mesh=mesh,
      scratch_types=[
          pltpu.VMEM((CH, C), jnp.int32),
          pltpu.VMEM((CH, C), jnp.int32),
          pltpu.VMEM((RQ, C, D), jnp.float32),
          pltpu.VMEM((RQ, C, D), jnp.float32),
      ] + [pltpu.SemaphoreType.DMA] * (2 * RQ),
  )
  def k(x_hbm, col_hbm, row_hbm, gc_hbm, gr_hbm, idxc, idxr, bufc, bufr,
        *sems):
    sem_g = sems[:RQ]
    sem_w = sems[RQ:]
    cid = lax.axis_index("c")
    sid = lax.axis_index("s")
    wid = sid * 2 + cid
    base = wid * EW
    pltpu.sync_copy(col_hbm.at[wid], idxc)
    pltpu.sync_copy(row_hbm.at[wid], idxr)

    def fire(chunk, b):
      pltpu.async_copy(x_hbm.at[idxc.at[chunk]], bufc.at[b], sem_g[b])
      pltpu.async_copy(x_hbm.at[idxr.at[chunk]], bufr.at[b], sem_g[b])

    def gc_dst(chunk):
      return gc_hbm.at[pl.ds(base + chunk * C, C)]

    def gr_dst(chunk):
      return gr_hbm.at[pl.ds(base + chunk * C, C)]

    def drain(chunk, b):
      pltpu.make_async_copy(x_hbm.at[idxc.at[chunk]], bufc.at[b],
                            sem_g[b]).wait()
      pltpu.make_async_copy(x_hbm.at[idxr.at[chunk]], bufr.at[b],
                            sem_g[b]).wait()
      pltpu.async_copy(bufc.at[b], gc_dst(chunk), sem_w[b])
      pltpu.async_copy(bufr.at[b], gr_dst(chunk), sem_w[b])

    def wait_writes(chunk, b):
      pltpu.make_async_copy(bufc.at[b], gc_dst(chunk), sem_w[b]).wait()
      pltpu.make_async_copy(bufr.at[b], gr_dst(chunk), sem_w[b]).wait()

    for b in range(RQ):
      fire(b, b)

    @pl.loop(0, CH - RQ, step=RQ)
    def _(i):
      for b in range(RQ):
        chunk = i + b
        drain(chunk, b)
        wait_writes(chunk, b)
        fire(chunk + RQ, b)

    for b in range(RQ):
      chunk = CH - RQ + b
      drain(chunk, b)
      wait_writes(chunk, b)

  return k


def _scatter_kernel(E, D, NP, NW, CH, C):
  mesh = plsc.VectorSubcoreMesh(core_axis_name="c", subcore_axis_name="s")
  EW = CH * C
  NS = 16
  rows_per_sub = NP // NS

  @functools.partial(
      pl.kernel,
      out_type=jax.ShapeDtypeStruct((2, NP, D), jnp.float32),
      mesh=mesh,
      scratch_types=[
          pltpu.VMEM((CH, C), jnp.int32),
          pltpu.VMEM((RQS, C, D), jnp.float32),
          pltpu.VMEM_SHARED((NP, D), jnp.float32),
      ] + [pltpu.SemaphoreType.DMA] * (2 * RQS),
  )
  def k(msg_hbm, row_hbm, init_hbm, part_hbm, idx, buf, acc, *sems):
    sem_l = sems[:RQS]
    sem_s = sems[RQS:]
    cid = lax.axis_index("c")
    sid = lax.axis_index("s")
    wid = sid * 2 + cid
    base = wid * EW
    pltpu.sync_copy(init_hbm.at[cid, pl.ds(sid * rows_per_sub, rows_per_sub)],
                    acc.at[pl.ds(sid * rows_per_sub, rows_per_sub)])
    pltpu.sync_copy(row_hbm.at[wid], idx)
    plsc.subcore_barrier()

    def fire(chunk, b):
      pltpu.async_copy(msg_hbm.at[pl.ds(base + chunk * C, C)], buf.at[b],
                       sem_l[b])

    def drain(chunk, b):
      pltpu.make_async_copy(msg_hbm.at[pl.ds(base + chunk * C, C)], buf.at[b],
                            sem_l[b]).wait()
      pltpu.async_copy(buf.at[b], acc.at[idx.at[chunk]], sem_s[b], add=True)

    def wait_add(chunk, b):
      pltpu.make_async_copy(buf.at[b], acc.at[idx.at[chunk]], sem_s[b]).wait()

    for b in range(RQS):
      fire(b, b)

    @pl.loop(0, CH - RQS, step=RQS)
    def _(i):
      for b in range(RQS):
        chunk = i + b
        drain(chunk, b)
        wait_add(chunk, b)
        fire(chunk + RQS, b)

    for b in range(RQS):
      drain(CH - RQS + b, b)
    for b in range(RQS):
      wait_add(CH - RQS + b, b)

    plsc.subcore_barrier()
    pltpu.sync_copy(acc.at[pl.ds(sid * rows_per_sub, rows_per_sub)],
                    part_hbm.at[cid, pl.ds(sid * rows_per_sub, rows_per_sub)])

  return k


def _tc_edge_body(gc_ref, gr_ref, offt2_ref, wb_ref, out_ref):
  xc = gc_ref[...]
  diff = xc - gr_ref[...]
  offt2 = offt2_ref[...]
  on2 = 0.25 * jnp.sum(offt2 * offt2, axis=0, keepdims=True)
  ones_m = jnp.ones((diff.shape[1], offt2.shape[1]), jnp.float32)
  t1 = lax.dot_general(diff * diff, ones_m, (((1,), (0,)), ((), ())),
                       preferred_element_type=jnp.float32)
  t2 = lax.dot_general(diff, offt2, (((1,), (0,)), ((), ())),
                       preferred_element_type=jnp.float32)
  d2 = jnp.maximum(t1 + t2 + on2, 0.0)
  sim = -jnp.sqrt(d2)
  sim = sim - jnp.max(sim, axis=1, keepdims=True)
  e = jnp.exp(sim)
  alpha = e / jnp.sum(e, axis=1, keepdims=True)
  mix = lax.dot_general(alpha, wb_ref[...], (((1,), (0,)), ((), ())),
                        preferred_element_type=jnp.float32)
  Dm = xc.shape[1]
  out_ref[...] = mix[:, :Dm] * xc + mix[:, Dm:]


def _combine_body(p_ref, o_ref):
  o_ref[...] = p_ref[0] + p_ref[1]


@jax.jit
def kernel(x, edge_index, offset, weight, bias):
  N, D = x.shape
  K = offset.shape[0]
  E = edge_index.shape[1]
  NW = 32
  CG = 80
  CS = 40
  G = 5
  assert E % (G * NW * CG) == 0 and E % (G * NW * CS) == 0
  ES = E // G
  CHG = ES // (NW * CG)
  CHS = ES // (NW * CS)
  assert CHG % RQ == 0 and CHG >= 2 * RQ
  assert CHS % RQS == 0 and CHS >= 2 * RQS

  row = edge_index[0]
  col = edge_index[1]
  row_g = row.reshape(G, NW, CHG, CG)
  col_g = col.reshape(G, NW, CHG, CG)
  row_s = row.reshape(G, NW, CHS, CS)

  offt2 = -2.0 * offset.T
  wb = jnp.concatenate([weight, bias], axis=1)
  BE = 1000
  assert ES % BE == 0
  NP = ((N + 1279) // 1280) * 1280

  gather_fn = _gather_kernel(ES, D, NW, CHG, CG)
  scatter_fn = _scatter_kernel(ES, D, NP, NW, CHS, CS)
  tc_fn = pl.pallas_call(
      _tc_edge_body,
      grid=(ES // BE,),
      in_specs=[
          pl.BlockSpec((BE, D), lambda i: (i, 0)),
          pl.BlockSpec((BE, D), lambda i: (i, 0)),
          pl.BlockSpec((D, K), lambda i: (0, 0)),
          pl.BlockSpec((K, 2 * D), lambda i: (0, 0)),
      ],
      out_specs=pl.BlockSpec((BE, D), lambda i: (i, 0)),
      out_shape=jax.ShapeDtypeStruct((ES, D), jnp.float32),
  )

  parts = jnp.zeros((2, NP, D), jnp.float32)
  for s in range(G):
    gc, gr = gather_fn(x, col_g[s], row_g[s])
    msg = tc_fn(gc, gr, offt2, wb)
    parts = scatter_fn(msg, row_s[s], parts)

  BN = 1280
  assert NP % BN == 0
  out = pl.pallas_call(
      _combine_body,
      grid=(NP // BN,),
      in_specs=[pl.BlockSpec((2, BN, D), lambda i: (0, i, 0))],
      out_specs=pl.BlockSpec((BN, D), lambda i: (i, 0)),
      out_shape=jax.ShapeDtypeStruct((NP, D), jnp.float32),
  )(parts)
  return out[:N]


# speedup vs baseline: 1.0547x; 1.0145x over previous
import functools

import jax
import jax.numpy as jnp
from jax import lax
from jax.experimental import pallas as pl
from jax.experimental.pallas import tpu as pltpu
from jax.experimental.pallas import tpu_sc as plsc

RQ = 5
RQS = 2


def _gather_kernel(E, D, NW, CH, C):
  mesh = plsc.VectorSubcoreMesh(core_axis_name="c", subcore_axis_name="s")
  EW = CH * C

  @functools.partial(
      pl.kernel,
      out_type=[
          jax.ShapeDtypeStruct((E, D), jnp.float32),
          jax.ShapeDtypeStruct((E, D), jnp.float32),
      ],
      mesh=mesh,
      scratch_types=[
          pltpu.VMEM((CH, C), jnp.int32),
          pltpu.VMEM((CH, C), jnp.int32),
          pltpu.VMEM((RQ, C, D), jnp.float32),
          pltpu.VMEM((RQ, C, D), jnp.float32),
      ] + [pltpu.SemaphoreType.DMA] * (2 * RQ),
  )
  def k(x_hbm, col_hbm, row_hbm, gc_hbm, gr_hbm, idxc, idxr, bufc, bufr,
        *sems):
    sem_g = sems[:RQ]
    sem_w = sems[RQ:]
    cid = lax.axis_index("c")
    sid = lax.axis_index("s")
    wid = sid * 2 + cid
    base = wid * EW
    pltpu.sync_copy(col_hbm.at[wid], idxc)
    pltpu.sync_copy(row_hbm.at[wid], idxr)

    def fire(chunk, b):
      pltpu.async_copy(x_hbm.at[idxc.at[chunk]], bufc.at[b], sem_g[b])
      pltpu.async_copy(x_hbm.at[idxr.at[chunk]], bufr.at[b], sem_g[b])

    def gc_dst(chunk):
      return gc_hbm.at[pl.ds(base + chunk * C, C)]

    def gr_dst(chunk):
      return gr_hbm.at[pl.ds(base + chunk * C, C)]

    def drain(chunk, b):
      pltpu.make_async_copy(x_hbm.at[idxc.at[chunk]], bufc.at[b],
                            sem_g[b]).wait()
      pltpu.make_async_copy(x_hbm.at[idxr.at[chunk]], bufr.at[b],
                            sem_g[b]).wait()
      pltpu.async_copy(bufc.at[b], gc_dst(chunk), sem_w[b])
      pltpu.async_copy(bufr.at[b], gr_dst(chunk), sem_w[b])

    def wait_writes(chunk, b):
      pltpu.make_async_copy(bufc.at[b], gc_dst(chunk), sem_w[b]).wait()
      pltpu.make_async_copy(bufr.at[b], gr_dst(chunk), sem_w[b]).wait()

    for b in range(RQ):
      fire(b, b)

    @pl.loop(0, CH - RQ, step=RQ)
    def _(i):
      for b in range(RQ):
        chunk = i + b
        drain(chunk, b)
        wait_writes(chunk, b)
        fire(chunk + RQ, b)

    for b in range(RQ):
      chunk = CH - RQ + b
      drain(chunk, b)
      wait_writes(chunk, b)

  return k


def _scatter_kernel(E, D, NP, NW, CH, C, zero_init):
  mesh = plsc.VectorSubcoreMesh(core_axis_name="c", subcore_axis_name="s")
  EW = CH * C
  NS = 16
  rows_per_sub = NP // NS

  def body(msg_hbm, row_hbm, *rest):
    if zero_init:
      init_hbm = None
      part_hbm, idx, buf, acc, *sems = rest
    else:
      init_hbm, part_hbm, idx, buf, acc, *sems = rest
    sem_l = sems[:RQS]
    sem_s = sems[RQS:]
    cid = lax.axis_index("c")
    sid = lax.axis_index("s")
    wid = sid * 2 + cid
    base = wid * EW
    if zero_init:
      z = jnp.zeros((16,), jnp.float32)
      for j in range(C):
        for l in range(D // 16):
          buf[0, j, pl.ds(l * 16, 16)] = z
      assert rows_per_sub % C == 0
      for j in range(rows_per_sub // C):
        pltpu.sync_copy(buf.at[0],
                        acc.at[pl.ds(sid * rows_per_sub + j * C, C)])
    else:
      pltpu.sync_copy(
          init_hbm.at[cid, pl.ds(sid * rows_per_sub, rows_per_sub)],
          acc.at[pl.ds(sid * rows_per_sub, rows_per_sub)])
    pltpu.sync_copy(row_hbm.at[wid], idx)
    plsc.subcore_barrier()

    def fire(chunk, b):
      pltpu.async_copy(msg_hbm.at[pl.ds(base + chunk * C, C)], buf.at[b],
                       sem_l[b])

    def drain(chunk, b):
      pltpu.make_async_copy(msg_hbm.at[pl.ds(base + chunk * C, C)], buf.at[b],
                            sem_l[b]).wait()
      pltpu.async_copy(buf.at[b], acc.at[idx.at[chunk]], sem_s[b], add=True)

    def wait_add(chunk, b):
      pltpu.make_async_copy(buf.at[b], acc.at[idx.at[chunk]], sem_s[b]).wait()

    for b in range(RQS):
      fire(b, b)

    @pl.loop(0, CH - RQS, step=RQS)
    def _(i):
      for b in range(RQS):
        chunk = i + b
        drain(chunk, b)
        wait_add(chunk, b)
        fire(chunk + RQS, b)

    for b in range(RQS):
      drain(CH - RQS + b, b)
    for b in range(RQS):
      wait_add(CH - RQS + b, b)

    plsc.subcore_barrier()
    pltpu.sync_copy(acc.at[pl.ds(sid * rows_per_sub, rows_per_sub)],
                    part_hbm.at[cid, pl.ds(sid * rows_per_sub, rows_per_sub)])

  return pl.kernel(
      body,
      out_type=jax.ShapeDtypeStruct((2, NP, D), jnp.float32),
      mesh=mesh,
      scratch_types=[
          pltpu.VMEM((CH, C), jnp.int32),
          pltpu.VMEM((RQS, C, D), jnp.float32),
          pltpu.VMEM_SHARED((NP, D), jnp.float32),
      ] + [pltpu.SemaphoreType.DMA] * (2 * RQS),
  )


def _tc_edge_body(gc_ref, gr_ref, offt2_ref, wb_ref, out_ref):
  xc = gc_ref[...]
  diff = xc - gr_ref[...]
  offt2 = offt2_ref[...]
  on2 = 0.25 * jnp.sum(offt2 * offt2, axis=0, keepdims=True)
  ones_m = jnp.ones((diff.shape[1], offt2.shape[1]), jnp.float32)
  t1 = lax.dot_general(diff * diff, ones_m, (((1,), (0,)), ((), ())),
                       preferred_element_type=jnp.float32)
  t2 = lax.dot_general(diff, offt2, (((1,), (0,)), ((), ())),
                       preferred_element_type=jnp.float32)
  d2 = jnp.maximum(t1 + t2 + on2, 0.0)
  sim = -jnp.sqrt(d2)
  sim = sim - jnp.max(sim, axis=1, keepdims=True)
  e = jnp.exp(sim)
  alpha = e / jnp.sum(e, axis=1, keepdims=True)
  mix = lax.dot_general(alpha, wb_ref[...], (((1,), (0,)), ((), ())),
                        preferred_element_type=jnp.float32)
  Dm = xc.shape[1]
  out_ref[...] = mix[:, :Dm] * xc + mix[:, Dm:]


def _combine_body(p_ref, o_ref):
  o_ref[...] = p_ref[0] + p_ref[1]


@jax.jit
def kernel(x, edge_index, offset, weight, bias):
  N, D = x.shape
  K = offset.shape[0]
  E = edge_index.shape[1]
  NW = 32
  CG = 80
  CS = 40
  G = 5
  assert E % (G * NW * CG) == 0 and E % (G * NW * CS) == 0
  ES = E // G
  CHG = ES // (NW * CG)
  CHS = ES // (NW * CS)
  assert CHG % RQ == 0 and CHG >= 2 * RQ
  assert CHS % RQS == 0 and CHS >= 2 * RQS

  row = edge_index[0]
  col = edge_index[1]
  row_g = row.reshape(G, NW, CHG, CG)
  col_g = col.reshape(G, NW, CHG, CG)
  row_s = row.reshape(G, NW, CHS, CS)

  offt2 = -2.0 * offset.T
  wb = jnp.concatenate([weight, bias], axis=1)
  BE = 1000
  assert ES % BE == 0
  NP = ((N + 1279) // 1280) * 1280

  gather_fn = _gather_kernel(ES, D, NW, CHG, CG)
  scatter0_fn = _scatter_kernel(ES, D, NP, NW, CHS, CS, True)
  scatter_fn = _scatter_kernel(ES, D, NP, NW, CHS, CS, False)
  tc_fn = pl.pallas_call(
      _tc_edge_body,
      grid=(ES // BE,),
      in_specs=[
          pl.BlockSpec((BE, D), lambda i: (i, 0)),
          pl.BlockSpec((BE, D), lambda i: (i, 0)),
          pl.BlockSpec((D, K), lambda i: (0, 0)),
          pl.BlockSpec((K, 2 * D), lambda i: (0, 0)),
      ],
      out_specs=pl.BlockSpec((BE, D), lambda i: (i, 0)),
      out_shape=jax.ShapeDtypeStruct((ES, D), jnp.float32),
  )

  parts = None
  for s in range(G):
    gc, gr = gather_fn(x, col_g[s], row_g[s])
    msg = tc_fn(gc, gr, offt2, wb)
    if parts is None:
      parts = scatter0_fn(msg, row_s[s])
    else:
      parts = scatter_fn(msg, row_s[s], parts)

  BN = 1000
  assert N % BN == 0
  out = pl.pallas_call(
      _combine_body,
      grid=(N // BN,),
      in_specs=[pl.BlockSpec((2, BN, D), lambda i: (0, i, 0))],
      out_specs=pl.BlockSpec((BN, D), lambda i: (i, 0)),
      out_shape=jax.ShapeDtypeStruct((N, D), jnp.float32),
  )(parts)
  return out
